# baseline (speedup 1.0000x reference)
import functools

import jax
import jax.numpy as jnp
from jax import lax
from jax.experimental import pallas as pl
from jax.experimental.pallas import tpu as pltpu
from jax.experimental.pallas import tpu_sc as plsc

GRID_H, GRID_W, EMBED_DIM = 32, 32, 768
D = EMBED_DIM // 3
NUM_CORES = 2
NUM_SUBCORES = 16
NW = NUM_CORES * NUM_SUBCORES
ROWS_PER_W = (GRID_H * GRID_W) // NW
N_OUT = GRID_H * GRID_W + 1


def _pos_emb_body(row_hbm, col_hbm, time_hbm, cls_hbm, out_hbm,
                  rtab_v, buf_v, sem_o, sem_g, sem_c, sem_t):
    wid = lax.axis_index("s") * NUM_CORES + lax.axis_index("c")
    base = wid * ROWS_PER_W
    cp_g = pltpu.async_copy(row_hbm, rtab_v, sem_g)
    cp_c = pltpu.async_copy(col_hbm, buf_v.at[:, pl.ds(D, D)], sem_c)
    cp_t = pltpu.async_copy(time_hbm.at[pl.ds(base, ROWS_PER_W)],
                            buf_v.at[:, pl.ds(2 * D, D)], sem_t)

    @pl.when(wid == 0)
    def _():
        pltpu.sync_copy(cls_hbm, out_hbm.at[pl.ds(0, 1), 0])

    cp_g.wait()
    for j in range(D // 16):
        v = rtab_v[wid, pl.ds(16 * j, 16)]
        for r in range(ROWS_PER_W):
            buf_v[r, pl.ds(16 * j, 16)] = v
    cp_c.wait()
    cp_t.wait()
    cp_o = pltpu.async_copy(buf_v, out_hbm.at[pl.ds(1 + base, ROWS_PER_W), 0],
                            sem_o)
    cp_o.wait()


_pos_emb = functools.partial(
    pl.kernel,
    mesh=plsc.VectorSubcoreMesh(core_axis_name="c", subcore_axis_name="s"),
    out_type=jax.ShapeDtypeStruct((N_OUT, 1, EMBED_DIM), jnp.float32),
    scratch_types=[
        pltpu.VMEM((GRID_H, D), jnp.float32),
        pltpu.VMEM((ROWS_PER_W, EMBED_DIM), jnp.float32),
        pltpu.SemaphoreType.DMA,
        pltpu.SemaphoreType.DMA,
        pltpu.SemaphoreType.DMA,
        pltpu.SemaphoreType.DMA,
    ],
)(_pos_emb_body)


def kernel(x, row_embed, col_embed, time_embed, cls_token_pos):
    del x
    out = _pos_emb(row_embed, col_embed, time_embed,
                   cls_token_pos.reshape(1, EMBED_DIM))
    return out.reshape(1, N_OUT, EMBED_DIM)


# speedup vs baseline: 1.0319x; 1.0319x over previous
import functools

import jax
import jax.numpy as jnp
from jax import lax
from jax.experimental import pallas as pl
from jax.experimental.pallas import tpu as pltpu
from jax.experimental.pallas import tpu_sc as plsc

GRID_H, GRID_W, EMBED_DIM = 32, 32, 768
D = EMBED_DIM // 3
NUM_CORES = 2
NUM_SUBCORES = 16
NW = NUM_CORES * NUM_SUBCORES
ROWS_PER_W = (GRID_H * GRID_W) // NW
N_OUT = GRID_H * GRID_W + 1


def _pos_emb_body(row_hbm, col_hbm, time_hbm, cls_hbm, out_hbm,
                  idx_r, row_v, col_v, time_v, sem_o, sem_g, sem_c, sem_t):
    wid = lax.axis_index("s") * NUM_CORES + lax.axis_index("c")
    base = wid * ROWS_PER_W
    widv = jnp.full((16,), wid, dtype=jnp.int32)
    idx_r[pl.ds(0, 16)] = widv
    idx_r[pl.ds(16, 16)] = widv
    cp_g = pltpu.async_copy(row_hbm.at[idx_r], row_v, sem_g)
    cp_ci = pltpu.async_copy(col_hbm, col_v, sem_c)
    cp_ti = pltpu.async_copy(time_hbm.at[pl.ds(base, ROWS_PER_W)], time_v,
                             sem_t)
    out_rows = out_hbm.at[pl.ds(1 + base, ROWS_PER_W), 0]
    cp_ci.wait()
    cp_c = pltpu.async_copy(col_v, out_rows.at[:, pl.ds(D, D)], sem_o)
    cp_ti.wait()
    cp_t = pltpu.async_copy(time_v, out_rows.at[:, pl.ds(2 * D, D)], sem_o)
    cp_g.wait()
    cp_r = pltpu.async_copy(row_v, out_rows.at[:, pl.ds(0, D)], sem_o)

    @pl.when(wid == 0)
    def _():
        pltpu.sync_copy(cls_hbm, out_hbm.at[pl.ds(0, 1), 0])

    cp_c.wait()
    cp_t.wait()
    cp_r.wait()


_pos_emb = functools.partial(
    pl.kernel,
    mesh=plsc.VectorSubcoreMesh(core_axis_name="c", subcore_axis_name="s"),
    out_type=jax.ShapeDtypeStruct((N_OUT, 1, EMBED_DIM), jnp.float32),
    scratch_types=[
        pltpu.VMEM((ROWS_PER_W,), jnp.int32),
        pltpu.VMEM((ROWS_PER_W, D), jnp.float32),
        pltpu.VMEM((ROWS_PER_W, D), jnp.float32),
        pltpu.VMEM((ROWS_PER_W, D), jnp.float32),
        pltpu.SemaphoreType.DMA,
        pltpu.SemaphoreType.DMA,
        pltpu.SemaphoreType.DMA,
        pltpu.SemaphoreType.DMA,
    ],
)(_pos_emb_body)


def kernel(x, row_embed, col_embed, time_embed, cls_token_pos):
    del x
    out = _pos_emb(row_embed, col_embed, time_embed,
                   cls_token_pos.reshape(1, EMBED_DIM))
    return out.reshape(1, N_OUT, EMBED_DIM)
